# Initial kernel scaffold; baseline (speedup 1.0000x reference)
import jax
import jax.numpy as jnp
from jax.experimental import pallas as pl


def kernel(X, centroids):
    raise NotImplementedError("write your pallas kernel here")


# speedup vs baseline: 1.0667x; 1.0667x over previous
import jax
import jax.numpy as jnp
from jax.experimental import pallas as pl

_N = 100000
_D = 16
_K = 1024
_BN = 2000


def _assign_block(x_ref, c_ref, o_ref):
    x = x_ref[...]
    c = c_ref[...]
    dots = jax.lax.dot_general(
        x, c, (((1,), (1,)), ((), ())),
        preferred_element_type=jnp.float32,
    )
    a2 = jnp.sum(x * x, axis=1, keepdims=True)
    b2 = jnp.sum(c * c, axis=1)
    d2 = (a2 + b2[None, :]) - 2.0 * dots
    o_ref[0, 0, :] = jnp.argmin(d2, axis=1).astype(jnp.int32)


def kernel(X, centroids):
    grid = _N // _BN
    out = pl.pallas_call(
        _assign_block,
        grid=(grid,),
        in_specs=[
            pl.BlockSpec((_BN, _D), lambda i: (i, 0)),
            pl.BlockSpec((_K, _D), lambda i: (0, 0)),
        ],
        out_specs=pl.BlockSpec((1, 1, _BN), lambda i: (i, 0, 0)),
        out_shape=jax.ShapeDtypeStruct((grid, 1, _BN), jnp.int32),
    )(X, centroids)
    return out.reshape(_N)


# speedup vs baseline: 2.1890x; 2.0521x over previous
import jax
import jax.numpy as jnp
from jax.experimental import pallas as pl

_N = 100000
_D = 16
_K = 1024
_BN = 2000


def _assign_block(x_ref, ca_ref, o_ref):
    x = x_ref[...]
    ca = ca_ref[...]
    d2 = jax.lax.dot_general(
        ca, x, (((1,), (1,)), ((), ())),
        preferred_element_type=jnp.float32,
    )
    o_ref[0, 0, :] = jnp.argmin(d2, axis=0).astype(jnp.int32)


def kernel(X, centroids):
    grid = _N // _BN
    b2 = jnp.sum(centroids * centroids, axis=1)
    b2_hi = b2.astype(jnp.bfloat16).astype(jnp.float32)
    b2_lo = b2 - b2_hi
    ca = jnp.concatenate(
        [-2.0 * centroids, b2_hi[:, None], b2_lo[:, None]], axis=1
    )
    ones = jnp.ones((_N, 2), jnp.float32)
    xa = jnp.concatenate([X, ones], axis=1)
    out = pl.pallas_call(
        _assign_block,
        grid=(grid,),
        in_specs=[
            pl.BlockSpec((_BN, _D + 2), lambda i: (i, 0)),
            pl.BlockSpec((_K, _D + 2), lambda i: (0, 0)),
        ],
        out_specs=pl.BlockSpec((1, 1, _BN), lambda i: (i, 0, 0)),
        out_shape=jax.ShapeDtypeStruct((grid, 1, _BN), jnp.int32),
    )(xa, ca)
    return out.reshape(_N)


# speedup vs baseline: 2.6877x; 1.2279x over previous
import jax
import jax.numpy as jnp
from jax.experimental import pallas as pl

_N = 100000
_D = 16
_K = 1024
_DA = _D + 3
_BN = 2000


def _assign_block(x_ref, ca_ref, o_ref):
    x = x_ref[...]
    ca = ca_ref[...]
    d2 = jax.lax.dot_general(
        ca, x, (((1,), (1,)), ((), ())),
        preferred_element_type=jnp.float32,
    )
    o_ref[0, 0, :] = jnp.argmin(d2, axis=0).astype(jnp.int32)


def kernel(X, centroids):
    grid = _N // _BN
    b2 = jnp.sum(centroids * centroids, axis=1)
    mask = jnp.int32(-65536)

    def _trunc(v):
        return jax.lax.bitcast_convert_type(
            jax.lax.bitcast_convert_type(v, jnp.int32) & mask, jnp.float32)

    b2_hi = _trunc(b2)
    r = b2 - b2_hi
    b2_mid = _trunc(r)
    b2_lo = r - b2_mid
    ca = jnp.concatenate(
        [-2.0 * centroids, b2_hi[:, None], b2_mid[:, None], b2_lo[:, None]],
        axis=1,
    ).astype(jnp.bfloat16)
    ones = jnp.ones((_N, 3), jnp.float32)
    xa = jnp.concatenate([X, ones], axis=1).astype(jnp.bfloat16)
    out = pl.pallas_call(
        _assign_block,
        grid=(grid,),
        in_specs=[
            pl.BlockSpec((_BN, _DA), lambda i: (i, 0)),
            pl.BlockSpec((_K, _DA), lambda i: (0, 0)),
        ],
        out_specs=pl.BlockSpec((1, 1, _BN), lambda i: (i, 0, 0)),
        out_shape=jax.ShapeDtypeStruct((grid, 1, _BN), jnp.int32),
    )(xa, ca)
    return out.reshape(_N)


# speedup vs baseline: 2.8647x; 1.0659x over previous
import jax
import jax.numpy as jnp
from jax.experimental import pallas as pl

_N = 100000
_D = 16
_K = 1024
_DA = _D + 3
_BN = 5000


def _assign_block(x_ref, ca_ref, o_ref):
    x = x_ref[...]
    ca = ca_ref[...]
    d2 = jax.lax.dot_general(
        ca, x, (((1,), (1,)), ((), ())),
        preferred_element_type=jnp.float32,
    )
    o_ref[0, 0, :] = jnp.argmin(d2, axis=0).astype(jnp.int32)


def kernel(X, centroids):
    grid = _N // _BN
    b2 = jnp.sum(centroids * centroids, axis=1)
    mask = jnp.int32(-65536)

    def _trunc(v):
        return jax.lax.bitcast_convert_type(
            jax.lax.bitcast_convert_type(v, jnp.int32) & mask, jnp.float32)

    b2_hi = _trunc(b2)
    r = b2 - b2_hi
    b2_mid = _trunc(r)
    b2_lo = r - b2_mid
    ca = jnp.concatenate(
        [-2.0 * centroids, b2_hi[:, None], b2_mid[:, None], b2_lo[:, None]],
        axis=1,
    ).astype(jnp.bfloat16)
    ones = jnp.ones((_N, 3), jnp.float32)
    xa = jnp.concatenate([X, ones], axis=1).astype(jnp.bfloat16)
    out = pl.pallas_call(
        _assign_block,
        grid=(grid,),
        in_specs=[
            pl.BlockSpec((_BN, _DA), lambda i: (i, 0)),
            pl.BlockSpec((_K, _DA), lambda i: (0, 0)),
        ],
        out_specs=pl.BlockSpec((1, 1, _BN), lambda i: (i, 0, 0)),
        out_shape=jax.ShapeDtypeStruct((grid, 1, _BN), jnp.int32),
    )(xa, ca)
    return out.reshape(_N)


# speedup vs baseline: 3.0922x; 1.0794x over previous
import jax
import jax.numpy as jnp
from jax.experimental import pallas as pl

_N = 100000
_D = 16
_K = 1024
_DA = _D + 3
_BN = 5000


def _assign_block(x_ref, ca_ref, o_ref):
    xb = x_ref[...].astype(jnp.bfloat16)
    ones = jnp.ones((_BN, 3), jnp.bfloat16)
    xa = jnp.concatenate([xb, ones], axis=1)
    ca = ca_ref[...]
    d2 = jax.lax.dot_general(
        ca, xa, (((1,), (1,)), ((), ())),
        preferred_element_type=jnp.float32,
    )
    o_ref[0, 0, :] = jnp.argmin(d2, axis=0).astype(jnp.int32)


def kernel(X, centroids):
    grid = _N // _BN
    b2 = jnp.sum(centroids * centroids, axis=1)
    mask = jnp.int32(-65536)

    def _trunc(v):
        return jax.lax.bitcast_convert_type(
            jax.lax.bitcast_convert_type(v, jnp.int32) & mask, jnp.float32)

    b2_hi = _trunc(b2)
    r = b2 - b2_hi
    b2_mid = _trunc(r)
    b2_lo = r - b2_mid
    ca = jnp.concatenate(
        [-2.0 * centroids, b2_hi[:, None], b2_mid[:, None], b2_lo[:, None]],
        axis=1,
    ).astype(jnp.bfloat16)
    out = pl.pallas_call(
        _assign_block,
        grid=(grid,),
        in_specs=[
            pl.BlockSpec((_BN, _D), lambda i: (i, 0)),
            pl.BlockSpec((_K, _DA), lambda i: (0, 0)),
        ],
        out_specs=pl.BlockSpec((1, 1, _BN), lambda i: (i, 0, 0)),
        out_shape=jax.ShapeDtypeStruct((grid, 1, _BN), jnp.int32),
    )(X, ca)
    return out.reshape(_N)


# speedup vs baseline: 4.5229x; 1.4627x over previous
import jax
import jax.numpy as jnp
from jax.experimental import pallas as pl

_N = 100000
_D = 16
_K = 1024
_DA = _D + 3
_BN = 5120


def _assign_block(xt_ref, ca_ref, o_ref):
    xtb = xt_ref[...].astype(jnp.bfloat16)
    ones = jnp.ones((3, _BN), jnp.bfloat16)
    xat = jnp.concatenate([xtb, ones], axis=0)
    ca = ca_ref[...]
    d2 = jax.lax.dot_general(
        ca, xat, (((1,), (0,)), ((), ())),
        preferred_element_type=jnp.float32,
    )
    o_ref[0, 0, :] = jnp.argmin(d2, axis=0).astype(jnp.int32)


def kernel(X, centroids):
    grid = (_N + _BN - 1) // _BN
    b2 = jnp.sum(centroids * centroids, axis=1)
    mask = jnp.int32(-65536)

    def _trunc(v):
        return jax.lax.bitcast_convert_type(
            jax.lax.bitcast_convert_type(v, jnp.int32) & mask, jnp.float32)

    b2_hi = _trunc(b2)
    r = b2 - b2_hi
    b2_mid = _trunc(r)
    b2_lo = r - b2_mid
    ca = jnp.concatenate(
        [-2.0 * centroids, b2_hi[:, None], b2_mid[:, None], b2_lo[:, None]],
        axis=1,
    ).astype(jnp.bfloat16)
    out = pl.pallas_call(
        _assign_block,
        grid=(grid,),
        in_specs=[
            pl.BlockSpec((_D, _BN), lambda i: (0, i)),
            pl.BlockSpec((_K, _DA), lambda i: (0, 0)),
        ],
        out_specs=pl.BlockSpec((1, 1, _BN), lambda i: (i, 0, 0)),
        out_shape=jax.ShapeDtypeStruct((grid, 1, _BN), jnp.int32),
    )(X.T, ca)
    return out.reshape(grid * _BN)[:_N]


# speedup vs baseline: 4.6412x; 1.0261x over previous
import jax
import jax.numpy as jnp
from jax.experimental import pallas as pl

_N = 100000
_D = 16
_K = 1024
_DA = _D + 3
_BN = 10240


def _assign_block(xt_ref, ca_ref, o_ref):
    xtb = xt_ref[...].astype(jnp.bfloat16)
    ones = jnp.ones((3, _BN), jnp.bfloat16)
    xat = jnp.concatenate([xtb, ones], axis=0)
    ca = ca_ref[...]
    d2 = jax.lax.dot_general(
        ca, xat, (((1,), (0,)), ((), ())),
        preferred_element_type=jnp.float32,
    )
    o_ref[0, 0, :] = jnp.argmin(d2, axis=0).astype(jnp.int32)


def kernel(X, centroids):
    grid = (_N + _BN - 1) // _BN
    b2 = jnp.sum(centroids * centroids, axis=1)
    mask = jnp.int32(-65536)

    def _trunc(v):
        return jax.lax.bitcast_convert_type(
            jax.lax.bitcast_convert_type(v, jnp.int32) & mask, jnp.float32)

    b2_hi = _trunc(b2)
    r = b2 - b2_hi
    b2_mid = _trunc(r)
    b2_lo = r - b2_mid
    ca = jnp.concatenate(
        [-2.0 * centroids, b2_hi[:, None], b2_mid[:, None], b2_lo[:, None]],
        axis=1,
    ).astype(jnp.bfloat16)
    out = pl.pallas_call(
        _assign_block,
        grid=(grid,),
        in_specs=[
            pl.BlockSpec((_D, _BN), lambda i: (0, i)),
            pl.BlockSpec((_K, _DA), lambda i: (0, 0)),
        ],
        out_specs=pl.BlockSpec((1, 1, _BN), lambda i: (i, 0, 0)),
        out_shape=jax.ShapeDtypeStruct((grid, 1, _BN), jnp.int32),
    )(X.T, ca)
    return out.reshape(grid * _BN)[:_N]
